# baseline (speedup 1.0000x reference)
import functools

import jax
import jax.numpy as jnp
from jax import lax
from jax.experimental import pallas as pl
from jax.experimental.pallas import tpu as pltpu
from jax.experimental.pallas import tpu_sc as plsc

_N = 1600000
_TARGET_BER = 0.5
_NUM_CORES = 2
_NUM_SUBCORES = 16
_NUM_TILES = _NUM_CORES * _NUM_SUBCORES
_PER_TILE = _N // _NUM_TILES
_B = 400
_STEPS = _B // 16
_CHUNKS = _PER_TILE // _B

_mesh = plsc.VectorSubcoreMesh(
    core_axis_name="c", subcore_axis_name="s",
    num_cores=_NUM_CORES, num_subcores=_NUM_SUBCORES)


@functools.partial(
    pl.kernel,
    out_type=(
        jax.ShapeDtypeStruct((_NUM_TILES, 16), jnp.float32),
        jax.ShapeDtypeStruct((_NUM_TILES, 16), jnp.float32),
    ),
    mesh=_mesh,
    compiler_params=pltpu.CompilerParams(needs_layout_passes=False),
    scratch_types=[
        pltpu.VMEM((_B, 2), jnp.float32),
        pltpu.VMEM((_B,), jnp.int32),
        pltpu.VMEM((_B,), jnp.int32),
        pltpu.VMEM((16,), jnp.float32),
        pltpu.VMEM((16,), jnp.float32),
        pltpu.SemaphoreType.DMA,
    ],
)
def _partials_sc(in_hbm, tgt_hbm, sens_hbm, sums_hbm, cnts_hbm,
                 rows_v, tgt_v, sens_v, res_s_v, res_c_v, sem):
    wid = lax.axis_index("s") * _NUM_CORES + lax.axis_index("c")
    base = wid * _PER_TILE
    iota = lax.iota(jnp.int32, 16)
    zero = jnp.zeros((16,), jnp.float32)
    one = jnp.ones((16,), jnp.float32)

    def chunk_body(chunk, accs):
        start = base + chunk * _B
        cp_in = pltpu.async_copy(in_hbm.at[pl.ds(start, _B)], rows_v, sem)
        cp_t = pltpu.async_copy(tgt_hbm.at[pl.ds(start, _B)], tgt_v, sem)
        cp_s = pltpu.async_copy(sens_hbm.at[pl.ds(start, _B)], sens_v, sem)
        cp_in.wait()
        cp_t.wait()
        cp_s.wait()

        @plsc.parallel_loop(0, _STEPS, unroll=5, carry=accs)
        def accs(j, carry):
            s0, s1, s2, s3, c0, c1, c2, c3 = carry
            off = j * 16
            t = tgt_v[pl.ds(off, 16)]
            s = sens_v[pl.ds(off, 16)]
            x = plsc.load_gather(rows_v, [off + iota, t])
            x = jnp.abs(jnp.float32(1.0) - x)
            m0 = s == 0
            m1 = s == 1
            m2 = s == 2
            m3 = s == 3
            s0 = s0 + jnp.where(m0, x, zero)
            s1 = s1 + jnp.where(m1, x, zero)
            s2 = s2 + jnp.where(m2, x, zero)
            s3 = s3 + jnp.where(m3, x, zero)
            c0 = c0 + jnp.where(m0, one, zero)
            c1 = c1 + jnp.where(m1, one, zero)
            c2 = c2 + jnp.where(m2, one, zero)
            c3 = c3 + jnp.where(m3, one, zero)
            return (s0, s1, s2, s3, c0, c1, c2, c3)

        return accs

    accs = lax.fori_loop(
        0, _CHUNKS, chunk_body,
        (zero, zero, zero, zero, zero, zero, zero, zero))

    res_s = zero
    res_c = zero
    for g in range(4):
        res_s = jnp.where(iota == g, jnp.sum(accs[g]), res_s)
        res_c = jnp.where(iota == g, jnp.sum(accs[4 + g]), res_c)
    res_s_v[...] = res_s
    res_c_v[...] = res_c
    pltpu.sync_copy(res_s_v, sums_hbm.at[wid])
    pltpu.sync_copy(res_c_v, cnts_hbm.at[wid])


def _finalize_tc(s_ref, c_ref, o_ref):
    ts = jnp.sum(s_ref[...], axis=0, keepdims=True)
    tc = jnp.sum(c_ref[...], axis=0, keepdims=True)
    present = tc > 0
    means = jnp.where(present, ts / jnp.maximum(tc, jnp.float32(1e-12)), 0.0)
    li = lax.broadcasted_iota(jnp.int32, (1, 16), 1)
    ng = jnp.max(jnp.where(present, li + 1, 0)).astype(jnp.float32)
    gm = jnp.sum(means) / ng
    o_ref[...] = jnp.reshape(jnp.abs(jnp.float32(_TARGET_BER) - gm), (1, 1))


def _spin_tc(x_ref, o_ref):
    def body(i, y):
        return y * jnp.float32(0.9999) + jnp.float32(1e-6)
    o_ref[...] = lax.fori_loop(0, 20000, body, x_ref[...])


def kernel(input, target, sens):
    sums, cnts = _partials_sc(input, target, sens)
    spun = pl.pallas_call(
        _spin_tc,
        out_shape=jax.ShapeDtypeStruct((256, 256), jnp.float32),
    )(jnp.zeros((256, 256), jnp.float32))
    res = pl.pallas_call(
        _finalize_tc,
        out_shape=jax.ShapeDtypeStruct((1, 1), jnp.float32),
    )(sums, cnts)
    return res[0, 0] + jnp.float32(0.0) * spun[0, 0]


# speedup vs baseline: 1.6711x; 1.6711x over previous
import functools

import jax
import jax.numpy as jnp
from jax import lax
from jax.experimental import pallas as pl
from jax.experimental.pallas import tpu as pltpu
from jax.experimental.pallas import tpu_sc as plsc

_N = 1600000
_TARGET_BER = 0.5
_NUM_CORES = 2
_NUM_SUBCORES = 16
_NUM_TILES = _NUM_CORES * _NUM_SUBCORES
_PER_TILE = _N // _NUM_TILES
_B = 400
_STEPS = _B // 16
_CHUNKS = _PER_TILE // _B

_mesh = plsc.VectorSubcoreMesh(
    core_axis_name="c", subcore_axis_name="s",
    num_cores=_NUM_CORES, num_subcores=_NUM_SUBCORES)


@functools.partial(
    pl.kernel,
    out_type=(
        jax.ShapeDtypeStruct((_NUM_TILES, 16), jnp.float32),
        jax.ShapeDtypeStruct((_NUM_TILES, 16), jnp.float32),
    ),
    mesh=_mesh,
    compiler_params=pltpu.CompilerParams(
        needs_layout_passes=False, use_tc_tiling_on_sc=True),
    scratch_types=[
        pltpu.VMEM((_B, 2), jnp.float32),
        pltpu.VMEM((_B, 2), jnp.float32),
        pltpu.VMEM((_B,), jnp.int32),
        pltpu.VMEM((_B,), jnp.int32),
        pltpu.VMEM((_B,), jnp.int32),
        pltpu.VMEM((_B,), jnp.int32),
        pltpu.VMEM((16,), jnp.float32),
        pltpu.VMEM((16,), jnp.float32),
        pltpu.SemaphoreType.DMA,
        pltpu.SemaphoreType.DMA,
    ],
)
def _partials_sc(in_hbm, tgt_hbm, sens_hbm, sums_hbm, cnts_hbm,
                 rows_a, rows_b, tgt_a, tgt_b, sens_a, sens_b,
                 res_s_v, res_c_v, sem_a, sem_b):
    wid = lax.axis_index("s") * _NUM_CORES + lax.axis_index("c")
    base = wid * _PER_TILE
    iota = lax.iota(jnp.int32, 16)
    zero = jnp.zeros((16,), jnp.float32)
    one = jnp.ones((16,), jnp.float32)

    def issue(chunk, rows_v, tgt_v, sens_v, sem):
        start = base + chunk * _B
        pltpu.async_copy(in_hbm.at[pl.ds(start, _B)], rows_v, sem)
        pltpu.async_copy(tgt_hbm.at[pl.ds(start, _B)], tgt_v, sem)
        pltpu.async_copy(sens_hbm.at[pl.ds(start, _B)], sens_v, sem)

    def drain(rows_v, tgt_v, sens_v, sem):
        pltpu.make_async_copy(in_hbm.at[pl.ds(0, _B)], rows_v, sem).wait()
        pltpu.make_async_copy(tgt_hbm.at[pl.ds(0, _B)], tgt_v, sem).wait()
        pltpu.make_async_copy(sens_hbm.at[pl.ds(0, _B)], sens_v, sem).wait()

    def compute(rows_v, tgt_v, sens_v, accs):
        @plsc.parallel_loop(0, _STEPS, unroll=5, carry=accs)
        def accs(j, carry):
            s0, s1, s2, s3, c0, c1, c2, c3 = carry
            off = j * 16
            t = tgt_v[pl.ds(off, 16)]
            s = sens_v[pl.ds(off, 16)]
            x = plsc.load_gather(rows_v, [off + iota, t])
            x = jnp.abs(jnp.float32(1.0) - x)
            m0 = s == 0
            m1 = s == 1
            m2 = s == 2
            m3 = s == 3
            s0 = s0 + jnp.where(m0, x, zero)
            s1 = s1 + jnp.where(m1, x, zero)
            s2 = s2 + jnp.where(m2, x, zero)
            s3 = s3 + jnp.where(m3, x, zero)
            c0 = c0 + jnp.where(m0, one, zero)
            c1 = c1 + jnp.where(m1, one, zero)
            c2 = c2 + jnp.where(m2, one, zero)
            c3 = c3 + jnp.where(m3, one, zero)
            return (s0, s1, s2, s3, c0, c1, c2, c3)

        return accs

    issue(0, rows_a, tgt_a, sens_a, sem_a)

    def pair_body(k, accs):
        c = 2 * k
        drain(rows_a, tgt_a, sens_a, sem_a)
        issue(c + 1, rows_b, tgt_b, sens_b, sem_b)
        accs = compute(rows_a, tgt_a, sens_a, accs)
        drain(rows_b, tgt_b, sens_b, sem_b)
        issue(c + 2, rows_a, tgt_a, sens_a, sem_a)
        accs = compute(rows_b, tgt_b, sens_b, accs)
        return accs

    accs = lax.fori_loop(
        0, (_CHUNKS - 1) // 2, pair_body,
        (zero, zero, zero, zero, zero, zero, zero, zero))
    drain(rows_a, tgt_a, sens_a, sem_a)
    accs = compute(rows_a, tgt_a, sens_a, accs)

    res_s = zero
    res_c = zero
    for g in range(4):
        res_s = jnp.where(iota == g, jnp.sum(accs[g]), res_s)
        res_c = jnp.where(iota == g, jnp.sum(accs[4 + g]), res_c)
    res_s_v[...] = res_s
    res_c_v[...] = res_c
    pltpu.sync_copy(res_s_v, sums_hbm.at[wid])
    pltpu.sync_copy(res_c_v, cnts_hbm.at[wid])


def _finalize_tc(s_ref, c_ref, o_ref):
    ts = jnp.sum(s_ref[...], axis=0, keepdims=True)
    tc = jnp.sum(c_ref[...], axis=0, keepdims=True)
    present = tc > 0
    means = jnp.where(present, ts / jnp.maximum(tc, jnp.float32(1e-12)), 0.0)
    li = lax.broadcasted_iota(jnp.int32, (1, 16), 1)
    ng = jnp.max(jnp.where(present, li + 1, 0)).astype(jnp.float32)
    gm = jnp.sum(means) / ng
    o_ref[...] = jnp.reshape(jnp.abs(jnp.float32(_TARGET_BER) - gm), (1, 1))


def kernel(input, target, sens):
    sums, cnts = _partials_sc(input, target, sens)
    res = pl.pallas_call(
        _finalize_tc,
        out_shape=jax.ShapeDtypeStruct((1, 1), jnp.float32),
    )(sums, cnts)
    return res[0, 0]


# speedup vs baseline: 1.7107x; 1.0237x over previous
import functools

import jax
import jax.numpy as jnp
from jax import lax
from jax.experimental import pallas as pl
from jax.experimental.pallas import tpu as pltpu
from jax.experimental.pallas import tpu_sc as plsc

_N = 1600000
_TARGET_BER = 0.5
_NUM_CORES = 2
_NUM_SUBCORES = 16
_NUM_TILES = _NUM_CORES * _NUM_SUBCORES
_PER_TILE = _N // _NUM_TILES
_B = 2000
_B_ALN = 2176
_STEPS = _B // 16
_CHUNKS = _PER_TILE // _B
_BLK = 12800

_mesh = plsc.VectorSubcoreMesh(
    core_axis_name="c", subcore_axis_name="s",
    num_cores=_NUM_CORES, num_subcores=_NUM_SUBCORES)


def _depad_tc(x_ref, o_ref):
    o_ref[...] = x_ref[...].T


@functools.partial(
    pl.kernel,
    out_type=(
        jax.ShapeDtypeStruct((_NUM_TILES, 16), jnp.float32),
        jax.ShapeDtypeStruct((_NUM_TILES, 16), jnp.float32),
    ),
    mesh=_mesh,
    compiler_params=pltpu.CompilerParams(needs_layout_passes=False),
    scratch_types=[
        pltpu.VMEM((2, _B_ALN), jnp.float32),
        pltpu.VMEM((2, _B_ALN), jnp.float32),
        pltpu.VMEM((_B,), jnp.int32),
        pltpu.VMEM((_B,), jnp.int32),
        pltpu.VMEM((_B,), jnp.int32),
        pltpu.VMEM((_B,), jnp.int32),
        pltpu.VMEM((16,), jnp.float32),
        pltpu.VMEM((16,), jnp.float32),
        pltpu.SemaphoreType.DMA,
        pltpu.SemaphoreType.DMA,
    ],
)
def _partials_sc(in_hbm, tgt_hbm, sens_hbm, sums_hbm, cnts_hbm,
                 rows_a, rows_b, tgt_a, tgt_b, sens_a, sens_b,
                 res_s_v, res_c_v, sem_a, sem_b):
    wid = lax.axis_index("s") * _NUM_CORES + lax.axis_index("c")
    base = wid * _PER_TILE
    iota = lax.iota(jnp.int32, 16)
    zero = jnp.zeros((16,), jnp.float32)
    one = jnp.ones((16,), jnp.float32)

    def win_start(start):
        s0 = jnp.minimum((start // 128) * 128, _N - _B_ALN)
        return pl.multiple_of(s0, 128)

    def issue(chunk, rows_v, tgt_v, sens_v, sem):
        start = base + chunk * _B
        pltpu.async_copy(in_hbm.at[:, pl.ds(win_start(start), _B_ALN)], rows_v, sem)
        pltpu.async_copy(tgt_hbm.at[pl.ds(start, _B)], tgt_v, sem)
        pltpu.async_copy(sens_hbm.at[pl.ds(start, _B)], sens_v, sem)

    def drain(rows_v, tgt_v, sens_v, sem):
        pltpu.make_async_copy(in_hbm.at[:, pl.ds(0, _B_ALN)], rows_v, sem).wait()
        pltpu.make_async_copy(tgt_hbm.at[pl.ds(0, _B)], tgt_v, sem).wait()
        pltpu.make_async_copy(sens_hbm.at[pl.ds(0, _B)], sens_v, sem).wait()

    def compute(chunk, rows_v, tgt_v, sens_v, accs):
        start = base + chunk * _B
        delta = start - win_start(start)

        @plsc.parallel_loop(0, _STEPS, unroll=5, carry=accs)
        def accs(j, carry):
            s0, s1, s2, s3, c0, c1, c2, c3 = carry
            off = j * 16
            t = tgt_v[pl.ds(off, 16)]
            s = sens_v[pl.ds(off, 16)]
            x = plsc.load_gather(rows_v, [t, delta + off + iota])
            x = jnp.abs(jnp.float32(1.0) - x)
            m0 = s == 0
            m1 = s == 1
            m2 = s == 2
            m3 = s == 3
            s0 = s0 + jnp.where(m0, x, zero)
            s1 = s1 + jnp.where(m1, x, zero)
            s2 = s2 + jnp.where(m2, x, zero)
            s3 = s3 + jnp.where(m3, x, zero)
            c0 = c0 + jnp.where(m0, one, zero)
            c1 = c1 + jnp.where(m1, one, zero)
            c2 = c2 + jnp.where(m2, one, zero)
            c3 = c3 + jnp.where(m3, one, zero)
            return (s0, s1, s2, s3, c0, c1, c2, c3)

        return accs

    issue(0, rows_a, tgt_a, sens_a, sem_a)

    def pair_body(k, accs):
        c = 2 * k
        drain(rows_a, tgt_a, sens_a, sem_a)
        issue(c + 1, rows_b, tgt_b, sens_b, sem_b)
        accs = compute(c, rows_a, tgt_a, sens_a, accs)
        drain(rows_b, tgt_b, sens_b, sem_b)
        issue(c + 2, rows_a, tgt_a, sens_a, sem_a)
        accs = compute(c + 1, rows_b, tgt_b, sens_b, accs)
        return accs

    accs = lax.fori_loop(
        0, (_CHUNKS - 1) // 2, pair_body,
        (zero, zero, zero, zero, zero, zero, zero, zero))
    drain(rows_a, tgt_a, sens_a, sem_a)
    accs = compute(_CHUNKS - 1, rows_a, tgt_a, sens_a, accs)

    res_s = zero
    res_c = zero
    for g in range(4):
        res_s = jnp.where(iota == g, jnp.sum(accs[g]), res_s)
        res_c = jnp.where(iota == g, jnp.sum(accs[4 + g]), res_c)
    res_s_v[...] = res_s
    res_c_v[...] = res_c
    pltpu.sync_copy(res_s_v, sums_hbm.at[wid])
    pltpu.sync_copy(res_c_v, cnts_hbm.at[wid])


def _finalize_tc(s_ref, c_ref, o_ref):
    ts = jnp.sum(s_ref[...], axis=0, keepdims=True)
    tc = jnp.sum(c_ref[...], axis=0, keepdims=True)
    present = tc > 0
    means = jnp.where(present, ts / jnp.maximum(tc, jnp.float32(1e-12)), 0.0)
    li = lax.broadcasted_iota(jnp.int32, (1, 16), 1)
    ng = jnp.max(jnp.where(present, li + 1, 0)).astype(jnp.float32)
    gm = jnp.sum(means) / ng
    o_ref[...] = jnp.reshape(jnp.abs(jnp.float32(_TARGET_BER) - gm), (1, 1))


def kernel(input, target, sens):
    flat = pl.pallas_call(
        _depad_tc,
        grid=(_N // _BLK,),
        in_specs=[pl.BlockSpec((_BLK, 2), lambda i: (i, 0))],
        out_specs=pl.BlockSpec((2, _BLK), lambda i: (0, i)),
        out_shape=jax.ShapeDtypeStruct((2, _N), jnp.float32),
    )(input)
    sums, cnts = _partials_sc(flat, target, sens)
    res = pl.pallas_call(
        _finalize_tc,
        out_shape=jax.ShapeDtypeStruct((1, 1), jnp.float32),
    )(sums, cnts)
    return res[0, 0]


# speedup vs baseline: 25.1875x; 14.7238x over previous
import functools

import jax
import jax.numpy as jnp
from jax import lax
from jax.experimental import pallas as pl
from jax.experimental.pallas import tpu as pltpu
from jax.experimental.pallas import tpu_sc as plsc

_N = 1600000
_TARGET_BER = 0.5
_NUM_CORES = 2
_NUM_SUBCORES = 16
_NUM_TILES = _NUM_CORES * _NUM_SUBCORES
_PER_TILE = _N // _NUM_TILES
_B = 2000
_B_ALN = 2176
_STEPS = _B // 16
_CHUNKS = _PER_TILE // _B
_BLK = 12800

_mesh = plsc.VectorSubcoreMesh(
    core_axis_name="c", subcore_axis_name="s",
    num_cores=_NUM_CORES, num_subcores=_NUM_SUBCORES)


@functools.partial(
    pl.kernel,
    out_type=(
        jax.ShapeDtypeStruct((_NUM_TILES, 16), jnp.float32),
        jax.ShapeDtypeStruct((_NUM_TILES, 16), jnp.float32),
    ),
    mesh=_mesh,
    compiler_params=pltpu.CompilerParams(needs_layout_passes=False),
    scratch_types=[
        pltpu.VMEM((2, _B_ALN), jnp.float32),
        pltpu.VMEM((2, _B_ALN), jnp.float32),
        pltpu.VMEM((_B,), jnp.int32),
        pltpu.VMEM((_B,), jnp.int32),
        pltpu.VMEM((_B,), jnp.int32),
        pltpu.VMEM((_B,), jnp.int32),
        pltpu.VMEM((16,), jnp.float32),
        pltpu.VMEM((16,), jnp.float32),
        pltpu.SemaphoreType.DMA,
        pltpu.SemaphoreType.DMA,
    ],
)
def _partials_sc(in_hbm, tgt_hbm, sens_hbm, sums_hbm, cnts_hbm,
                 rows_a, rows_b, tgt_a, tgt_b, sens_a, sens_b,
                 res_s_v, res_c_v, sem_a, sem_b):
    wid = lax.axis_index("s") * _NUM_CORES + lax.axis_index("c")
    base = wid * _PER_TILE
    iota = lax.iota(jnp.int32, 16)
    zero = jnp.zeros((16,), jnp.float32)
    one = jnp.ones((16,), jnp.float32)

    def win_start(start):
        s0 = jnp.minimum((start // 128) * 128, _N - _B_ALN)
        return pl.multiple_of(s0, 128)

    def issue(chunk, rows_v, tgt_v, sens_v, sem):
        start = base + chunk * _B
        pltpu.async_copy(in_hbm.at[:, pl.ds(win_start(start), _B_ALN)], rows_v, sem)
        pltpu.async_copy(tgt_hbm.at[pl.ds(start, _B)], tgt_v, sem)
        pltpu.async_copy(sens_hbm.at[pl.ds(start, _B)], sens_v, sem)

    def drain(rows_v, tgt_v, sens_v, sem):
        pltpu.make_async_copy(in_hbm.at[:, pl.ds(0, _B_ALN)], rows_v, sem).wait()
        pltpu.make_async_copy(tgt_hbm.at[pl.ds(0, _B)], tgt_v, sem).wait()
        pltpu.make_async_copy(sens_hbm.at[pl.ds(0, _B)], sens_v, sem).wait()

    def compute(chunk, rows_v, tgt_v, sens_v, accs):
        start = base + chunk * _B
        delta = start - win_start(start)

        @plsc.parallel_loop(0, _STEPS, unroll=5, carry=accs)
        def accs(j, carry):
            s0, s1, s2, s3, c0, c1, c2, c3 = carry
            off = j * 16
            t = tgt_v[pl.ds(off, 16)]
            s = sens_v[pl.ds(off, 16)]
            x = plsc.load_gather(rows_v, [t, delta + off + iota])
            x = jnp.abs(jnp.float32(1.0) - x)
            m0 = s == 0
            m1 = s == 1
            m2 = s == 2
            m3 = s == 3
            s0 = s0 + jnp.where(m0, x, zero)
            s1 = s1 + jnp.where(m1, x, zero)
            s2 = s2 + jnp.where(m2, x, zero)
            s3 = s3 + jnp.where(m3, x, zero)
            c0 = c0 + jnp.where(m0, one, zero)
            c1 = c1 + jnp.where(m1, one, zero)
            c2 = c2 + jnp.where(m2, one, zero)
            c3 = c3 + jnp.where(m3, one, zero)
            return (s0, s1, s2, s3, c0, c1, c2, c3)

        return accs

    issue(0, rows_a, tgt_a, sens_a, sem_a)

    def pair_body(k, accs):
        c = 2 * k
        drain(rows_a, tgt_a, sens_a, sem_a)
        issue(c + 1, rows_b, tgt_b, sens_b, sem_b)
        accs = compute(c, rows_a, tgt_a, sens_a, accs)
        drain(rows_b, tgt_b, sens_b, sem_b)
        issue(c + 2, rows_a, tgt_a, sens_a, sem_a)
        accs = compute(c + 1, rows_b, tgt_b, sens_b, accs)
        return accs

    accs = lax.fori_loop(
        0, (_CHUNKS - 1) // 2, pair_body,
        (zero, zero, zero, zero, zero, zero, zero, zero))
    drain(rows_a, tgt_a, sens_a, sem_a)
    accs = compute(_CHUNKS - 1, rows_a, tgt_a, sens_a, accs)

    res_s = zero
    res_c = zero
    for g in range(4):
        res_s = jnp.where(iota == g, jnp.sum(accs[g]), res_s)
        res_c = jnp.where(iota == g, jnp.sum(accs[4 + g]), res_c)
    res_s_v[...] = res_s
    res_c_v[...] = res_c
    pltpu.sync_copy(res_s_v, sums_hbm.at[wid])
    pltpu.sync_copy(res_c_v, cnts_hbm.at[wid])


def _finalize_tc(s_ref, c_ref, o_ref):
    ts = jnp.sum(s_ref[...], axis=0, keepdims=True)
    tc = jnp.sum(c_ref[...], axis=0, keepdims=True)
    present = tc > 0
    means = jnp.where(present, ts / jnp.maximum(tc, jnp.float32(1e-12)), 0.0)
    li = lax.broadcasted_iota(jnp.int32, (1, 16), 1)
    ng = jnp.max(jnp.where(present, li + 1, 0)).astype(jnp.float32)
    gm = jnp.sum(means) / ng
    o_ref[...] = jnp.reshape(jnp.abs(jnp.float32(_TARGET_BER) - gm), (1, 1))


def kernel(input, target, sens):
    sums, cnts = _partials_sc(jnp.swapaxes(input, 0, 1), target, sens)
    res = pl.pallas_call(
        _finalize_tc,
        out_shape=jax.ShapeDtypeStruct((1, 1), jnp.float32),
    )(sums, cnts)
    return res[0, 0]
